# Initial kernel scaffold; baseline (speedup 1.0000x reference)
import jax
import jax.numpy as jnp
from jax.experimental import pallas as pl


def kernel(features, edge_index, W_self1, W_neigh1, b1, W_self2, W_neigh2, b2):
    raise NotImplementedError("write your pallas kernel here")


# speedup vs baseline: 5.5164x; 5.5164x over previous
import functools

import jax
import jax.numpy as jnp
from jax import lax
from jax.experimental import pallas as pl
from jax.experimental.pallas import tpu as pltpu
from jax.experimental.pallas import tpu_sc as plsc

N_NODES = 10000
D = 256
E = 160000
N_TILES = 16
EDGES_PER_TILE = E // N_TILES
CHUNK = 80
N_CHUNKS = EDGES_PER_TILE // CHUNK
ROW_CHUNKS = N_NODES // CHUNK
ROW_LOOPS = -(-ROW_CHUNKS // N_TILES)
DEG_PAD = 10240
DEG_CHUNK = 1024
DEG_ZCHUNK = DEG_PAD // N_TILES


def _make_sc_aggregate(with_deg: bool, groups: int):
    width = 128 // groups
    n_pass = 2 * groups
    if with_deg:
        out_type = [jax.ShapeDtypeStruct((n_pass, N_NODES, width), jnp.float32),
                    jax.ShapeDtypeStruct((DEG_PAD,), jnp.float32)]
    else:
        out_type = jax.ShapeDtypeStruct((n_pass, N_NODES, width), jnp.float32)

    mesh = plsc.VectorSubcoreMesh(core_axis_name="c", subcore_axis_name="s")

    scratch_types = [
        pltpu.VMEM((N_CHUNKS, CHUNK), jnp.int32),
        pltpu.VMEM((N_CHUNKS, CHUNK), jnp.int32),
        pltpu.VMEM((CHUNK, width), jnp.float32),
        pltpu.VMEM((CHUNK, width), jnp.float32),
        pltpu.VMEM((CHUNK,), jnp.float32),
        pltpu.VMEM((DEG_CHUNK,), jnp.float32),
        pltpu.VMEM_SHARED((N_NODES, width), jnp.float32),
        pltpu.VMEM_SHARED((DEG_PAD,), jnp.float32),
        pltpu.SemaphoreType.DMA,
        pltpu.SemaphoreType.DMA,
    ]

    @functools.partial(
        pl.kernel, mesh=mesh, out_type=out_type, scratch_types=scratch_types,
        compiler_params=pltpu.CompilerParams(use_tc_tiling_on_sc=False))
    def sc_aggregate(tbl_hbm, srcidx_hbm, dstidx_hbm, *rest):
        if with_deg:
            agg_hbm, deg_hbm = rest[0], rest[1]
            scr = rest[2:]
        else:
            agg_hbm = rest[0]
            scr = rest[1:]
        (src_v, dst_v, buf0, buf1, ones_v, dzero_v, acc, deg_acc,
         gsem0, gsem1) = scr
        c = lax.axis_index("c")
        s = lax.axis_index("s")

        pltpu.sync_copy(srcidx_hbm.at[s], src_v)
        pltpu.sync_copy(dstidx_hbm.at[s], dst_v)

        for k in range(CHUNK // 16):
            ones_v[pl.ds(k * 16, 16)] = jnp.ones((16,), jnp.float32)

        def dz_step(i, carry):
            dzero_v[pl.ds(i * 16, 16)] = jnp.zeros((16,), jnp.float32)
            return carry

        lax.fori_loop(0, DEG_CHUNK // 16, dz_step, 0)

        def add_to_src(delta):
            def addc_step(i, carry):
                for k in range(CHUNK // 16):
                    sl = pl.ds(k * 16, 16)
                    src_v[i, sl] = src_v[i, sl] + delta
                return carry

            lax.fori_loop(0, N_CHUNKS, addc_step, 0)

        def gather(j, buf, sem):
            return pltpu.make_async_copy(tbl_hbm.at[src_v.at[j]], buf, sem)

        for g in range(groups):
            first = g == 0
            add_to_src(groups * c if first else 1)

            def zbuf_step(i, carry):
                for k in range(width // 16):
                    buf0[i, pl.ds(k * 16, 16)] = jnp.zeros((16,), jnp.float32)
                return carry

            lax.fori_loop(0, CHUNK, zbuf_step, 0)

            def zero_step(k, carry):
                i = s + k * N_TILES

                @pl.when(i < ROW_CHUNKS)
                def _():
                    pltpu.sync_copy(buf0, acc.at[pl.ds(i * CHUNK, CHUNK)])
                return carry

            lax.fori_loop(0, ROW_LOOPS, zero_step, 0)
            if with_deg and first:
                @pl.when(c == 0)
                def _():
                    pltpu.sync_copy(
                        dzero_v.at[pl.ds(0, DEG_ZCHUNK)],
                        deg_acc.at[pl.ds(s * DEG_ZCHUNK, DEG_ZCHUNK)])
            plsc.subcore_barrier()

            def scatter(j, buf):
                pltpu.sync_copy(buf, acc.at[dst_v.at[j]], add=True)
                if with_deg and first:
                    @pl.when(c == 0)
                    def _():
                        pltpu.sync_copy(ones_v, deg_acc.at[dst_v.at[j]],
                                        add=True)

            gather(0, buf0, gsem0).start()
            gather(1, buf1, gsem1).start()

            def step(t, carry):
                j = 2 * t
                gather(j, buf0, gsem0).wait()
                scatter(j, buf0)
                gather(j + 2, buf0, gsem0).start()
                gather(j + 1, buf1, gsem1).wait()
                scatter(j + 1, buf1)

                @pl.when(j + 3 < N_CHUNKS)
                def _():
                    gather(j + 3, buf1, gsem1).start()
                return carry

            lax.fori_loop(0, (N_CHUNKS - 1) // 2, step, 0)
            gather(N_CHUNKS - 1, buf0, gsem0).wait()
            scatter(N_CHUNKS - 1, buf0)

            plsc.subcore_barrier()

            def out_step(k, carry):
                i = s + k * N_TILES

                @pl.when(i < ROW_CHUNKS)
                def _():
                    rows = pl.ds(i * CHUNK, CHUNK)
                    pltpu.sync_copy(acc.at[rows],
                                    agg_hbm.at[groups * c + g, rows])
                return carry

            lax.fori_loop(0, ROW_LOOPS, out_step, 0)
            if with_deg and first:
                @pl.when(jnp.logical_and(c == 0, s < DEG_PAD // DEG_CHUNK))
                def _():
                    sl = pl.ds(s * DEG_CHUNK, DEG_CHUNK)
                    pltpu.sync_copy(deg_acc.at[sl], dzero_v)
                    pltpu.sync_copy(dzero_v, deg_hbm.at[sl])

    return sc_aggregate


@functools.lru_cache(maxsize=None)
def _sc_aggregate_l1():
    return _make_sc_aggregate(with_deg=True, groups=2)


@functools.lru_cache(maxsize=None)
def _sc_aggregate_l2():
    return _make_sc_aggregate(with_deg=False, groups=2)


BM = 1000


def _tc_layer_body(h_ref, agg_ref, deg_ref, ws_ref, wn_ref, b_ref, out_ref):
    inv = 1.0 / jnp.maximum(deg_ref[...], 1.0)
    h_neigh = agg_ref[...] * inv
    out = (jnp.dot(h_ref[...], ws_ref[...], preferred_element_type=jnp.float32)
           + jnp.dot(h_neigh, wn_ref[...], preferred_element_type=jnp.float32)
           + b_ref[...])
    out_ref[...] = jnp.maximum(out, 0.0)


@functools.lru_cache(maxsize=None)
def _tc_layer():
    return pl.pallas_call(
        _tc_layer_body,
        grid=(N_NODES // BM,),
        in_specs=[
            pl.BlockSpec((BM, D), lambda i: (i, 0)),
            pl.BlockSpec((BM, D), lambda i: (i, 0)),
            pl.BlockSpec((BM, 1), lambda i: (i, 0)),
            pl.BlockSpec((D, D), lambda i: (0, 0)),
            pl.BlockSpec((D, D), lambda i: (0, 0)),
            pl.BlockSpec((1, D), lambda i: (0, 0)),
        ],
        out_specs=pl.BlockSpec((BM, D), lambda i: (i, 0)),
        out_shape=jax.ShapeDtypeStruct((N_NODES, D), jnp.float32),
    )


def kernel(features, edge_index, W_self1, W_neigh1, b1, W_self2, W_neigh2, b2):
    src = edge_index[0].astype(jnp.int32)
    dst = edge_index[1].astype(jnp.int32)
    srcidx = (src * 4).reshape(N_TILES, N_CHUNKS, CHUNK)
    dstidx = dst.reshape(N_TILES, N_CHUNKS, CHUNK)

    tc = _tc_layer()
    agg1, deg = _sc_aggregate_l1()(
        features.reshape(4 * N_NODES, 64), srcidx, dstidx)
    agg1 = agg1.transpose(1, 0, 2).reshape(N_NODES, D)
    deg2d = deg[:N_NODES].reshape(N_NODES, 1)
    h1 = tc(features, agg1, deg2d, W_self1, W_neigh1, b1.reshape(1, D))
    agg2 = _sc_aggregate_l2()(
        h1.reshape(4 * N_NODES, 64), srcidx, dstidx)
    agg2 = agg2.transpose(1, 0, 2).reshape(N_NODES, D)
    h2 = tc(h1, agg2, deg2d, W_self2, W_neigh2, b2.reshape(1, D))
    return h2


# speedup vs baseline: 6.0137x; 1.0902x over previous
import functools

import jax
import jax.numpy as jnp
from jax import lax
from jax.experimental import pallas as pl
from jax.experimental.pallas import tpu as pltpu
from jax.experimental.pallas import tpu_sc as plsc

N_NODES = 10000
D = 256
E = 160000
N_TILES = 16
EDGES_PER_TILE = E // N_TILES
CHUNK = 80
N_CHUNKS = EDGES_PER_TILE // CHUNK
ROW_CHUNKS = N_NODES // CHUNK
ROW_LOOPS = -(-ROW_CHUNKS // N_TILES)
DEG_PAD = 10240
DEG_CHUNK = 1024
DEG_ZCHUNK = DEG_PAD // N_TILES


def _make_sc_aggregate(with_deg: bool, groups: int):
    width = 128 // groups
    n_pass = 2 * groups
    if with_deg:
        out_type = [jax.ShapeDtypeStruct((n_pass, N_NODES, width), jnp.float32),
                    jax.ShapeDtypeStruct((DEG_PAD,), jnp.float32)]
    else:
        out_type = jax.ShapeDtypeStruct((n_pass, N_NODES, width), jnp.float32)

    mesh = plsc.VectorSubcoreMesh(core_axis_name="c", subcore_axis_name="s")

    scratch_types = [
        pltpu.VMEM((N_CHUNKS, CHUNK), jnp.int32),
        pltpu.VMEM((N_CHUNKS, CHUNK), jnp.int32),
        pltpu.VMEM((CHUNK, width), jnp.float32),
        pltpu.VMEM((CHUNK, width), jnp.float32),
        pltpu.VMEM((CHUNK, width), jnp.float32),
        pltpu.VMEM((CHUNK, width), jnp.float32),
        pltpu.VMEM((CHUNK,), jnp.float32),
        pltpu.VMEM((DEG_CHUNK,), jnp.float32),
        pltpu.VMEM_SHARED((N_NODES, width), jnp.float32),
        pltpu.VMEM_SHARED((DEG_PAD,), jnp.float32),
        pltpu.SemaphoreType.DMA,
        pltpu.SemaphoreType.DMA,
        pltpu.SemaphoreType.DMA,
        pltpu.SemaphoreType.DMA,
        pltpu.SemaphoreType.DMA,
        pltpu.SemaphoreType.DMA,
        pltpu.SemaphoreType.DMA,
        pltpu.SemaphoreType.DMA,
        pltpu.SemaphoreType.DMA,
    ]

    @functools.partial(
        pl.kernel, mesh=mesh, out_type=out_type, scratch_types=scratch_types,
        compiler_params=pltpu.CompilerParams(use_tc_tiling_on_sc=False))
    def sc_aggregate(tbl_hbm, srcidx_hbm, dstidx_hbm, *rest):
        if with_deg:
            agg_hbm, deg_hbm = rest[0], rest[1]
            scr = rest[2:]
        else:
            agg_hbm = rest[0]
            scr = rest[1:]
        (src_v, dst_v, buf0, buf1, buf2, buf3, ones_v, dzero_v, acc, deg_acc,
         gsem0, gsem1, gsem2, gsem3, ssem0, ssem1, ssem2, ssem3, dsem) = scr
        bufs = (buf0, buf1, buf2, buf3)
        gsems = (gsem0, gsem1, gsem2, gsem3)
        ssems = (ssem0, ssem1, ssem2, ssem3)
        c = lax.axis_index("c")
        s = lax.axis_index("s")

        pltpu.sync_copy(srcidx_hbm.at[s], src_v)
        pltpu.sync_copy(dstidx_hbm.at[s], dst_v)

        for k in range(CHUNK // 16):
            ones_v[pl.ds(k * 16, 16)] = jnp.ones((16,), jnp.float32)

        def dz_step(i, carry):
            dzero_v[pl.ds(i * 16, 16)] = jnp.zeros((16,), jnp.float32)
            return carry

        lax.fori_loop(0, DEG_CHUNK // 16, dz_step, 0)

        def add_to_src(delta):
            def addc_step(i, carry):
                for k in range(CHUNK // 16):
                    sl = pl.ds(k * 16, 16)
                    src_v[i, sl] = src_v[i, sl] + delta
                return carry

            lax.fori_loop(0, N_CHUNKS, addc_step, 0)

        def gather(j, buf, sem):
            return pltpu.make_async_copy(tbl_hbm.at[src_v.at[j]], buf, sem)

        for g in range(groups):
            first = g == 0
            add_to_src(groups * c if first else 1)

            def zbuf_step(i, carry):
                for k in range(width // 16):
                    buf0[i, pl.ds(k * 16, 16)] = jnp.zeros((16,), jnp.float32)
                return carry

            lax.fori_loop(0, CHUNK, zbuf_step, 0)

            def zero_step(k, carry):
                i = s + k * N_TILES

                @pl.when(i < ROW_CHUNKS)
                def _():
                    pltpu.sync_copy(buf0, acc.at[pl.ds(i * CHUNK, CHUNK)])
                return carry

            lax.fori_loop(0, ROW_LOOPS, zero_step, 0)
            if with_deg and first:
                @pl.when(c == 0)
                def _():
                    pltpu.sync_copy(
                        dzero_v.at[pl.ds(0, DEG_ZCHUNK)],
                        deg_acc.at[pl.ds(s * DEG_ZCHUNK, DEG_ZCHUNK)])
            plsc.subcore_barrier()

            do_deg = with_deg and first

            def scat_start(j, buf, sem):
                pltpu.async_copy(buf, acc.at[dst_v.at[j]], sem, add=True)

            def scat_wait(buf, sem):
                pltpu.make_async_copy(buf, acc.at[dst_v.at[0]], sem).wait()

            def deg_start(j):
                pltpu.async_copy(ones_v, deg_acc.at[dst_v.at[j]], dsem,
                                 add=True)

            def deg_wait():
                pltpu.make_async_copy(ones_v, deg_acc.at[dst_v.at[0]],
                                      dsem).wait()

            gather(0, bufs[0], gsems[0]).start()
            gather(1, bufs[1], gsems[1]).start()

            def quad(t, carry):
                for b in range(4):
                    j = 4 * t + b
                    gather(j, bufs[b], gsems[b]).wait()
                    scat_start(j, bufs[b], ssems[b])
                    if do_deg:
                        @pl.when(c == 0)
                        def _():
                            deg_start(j)

                            @pl.when(t > 0)
                            def _():
                                deg_wait()
                    b2 = (b + 2) % 4
                    if b < 2:
                        @pl.when(t > 0)
                        def _():
                            scat_wait(bufs[b2], ssems[b2])
                        gather(j + 2, bufs[b2], gsems[b2]).start()
                    elif b == 2:
                        scat_wait(bufs[b2], ssems[b2])
                        gather(j + 2, bufs[b2], gsems[b2]).start()
                    else:
                        scat_wait(bufs[b2], ssems[b2])

                        @pl.when(t < (N_CHUNKS - 1) // 4 - 1)
                        def _():
                            gather(j + 2, bufs[b2], gsems[b2]).start()
                return carry

            lax.fori_loop(0, (N_CHUNKS - 1) // 4, quad, 0)
            gather(N_CHUNKS - 1, bufs[0], gsems[0]).wait()
            scat_start(N_CHUNKS - 1, bufs[0], ssems[0])
            scat_wait(bufs[2], ssems[2])
            scat_wait(bufs[3], ssems[3])
            scat_wait(bufs[0], ssems[0])
            if do_deg:
                @pl.when(c == 0)
                def _():
                    deg_start(N_CHUNKS - 1)
                    for _ in range(5):
                        deg_wait()

            plsc.subcore_barrier()

            def out_step(k, carry):
                i = s + k * N_TILES

                @pl.when(i < ROW_CHUNKS)
                def _():
                    rows = pl.ds(i * CHUNK, CHUNK)
                    pltpu.sync_copy(acc.at[rows],
                                    agg_hbm.at[groups * c + g, rows])
                return carry

            lax.fori_loop(0, ROW_LOOPS, out_step, 0)
            if with_deg and first:
                @pl.when(jnp.logical_and(c == 0, s < DEG_PAD // DEG_CHUNK))
                def _():
                    sl = pl.ds(s * DEG_CHUNK, DEG_CHUNK)
                    pltpu.sync_copy(deg_acc.at[sl], dzero_v)
                    pltpu.sync_copy(dzero_v, deg_hbm.at[sl])

    return sc_aggregate


@functools.lru_cache(maxsize=None)
def _sc_aggregate_l1():
    return _make_sc_aggregate(with_deg=True, groups=2)


@functools.lru_cache(maxsize=None)
def _sc_aggregate_l2():
    return _make_sc_aggregate(with_deg=False, groups=2)


BM = 1000


def _tc_layer_body(h_ref, agg_ref, deg_ref, ws_ref, wn_ref, b_ref, out_ref):
    inv = 1.0 / jnp.maximum(deg_ref[...], 1.0)
    h_neigh = agg_ref[...] * inv
    out = (jnp.dot(h_ref[...], ws_ref[...], preferred_element_type=jnp.float32)
           + jnp.dot(h_neigh, wn_ref[...], preferred_element_type=jnp.float32)
           + b_ref[...])
    out_ref[...] = jnp.maximum(out, 0.0)


@functools.lru_cache(maxsize=None)
def _tc_layer():
    return pl.pallas_call(
        _tc_layer_body,
        grid=(N_NODES // BM,),
        in_specs=[
            pl.BlockSpec((BM, D), lambda i: (i, 0)),
            pl.BlockSpec((BM, D), lambda i: (i, 0)),
            pl.BlockSpec((BM, 1), lambda i: (i, 0)),
            pl.BlockSpec((D, D), lambda i: (0, 0)),
            pl.BlockSpec((D, D), lambda i: (0, 0)),
            pl.BlockSpec((1, D), lambda i: (0, 0)),
        ],
        out_specs=pl.BlockSpec((BM, D), lambda i: (i, 0)),
        out_shape=jax.ShapeDtypeStruct((N_NODES, D), jnp.float32),
    )


def kernel(features, edge_index, W_self1, W_neigh1, b1, W_self2, W_neigh2, b2):
    src = edge_index[0].astype(jnp.int32)
    dst = edge_index[1].astype(jnp.int32)
    srcidx = (src * 4).reshape(N_TILES, N_CHUNKS, CHUNK)
    dstidx = dst.reshape(N_TILES, N_CHUNKS, CHUNK)

    tc = _tc_layer()
    agg1, deg = _sc_aggregate_l1()(
        features.reshape(4 * N_NODES, 64), srcidx, dstidx)
    agg1 = agg1.transpose(1, 0, 2).reshape(N_NODES, D)
    deg2d = deg[:N_NODES].reshape(N_NODES, 1)
    h1 = tc(features, agg1, deg2d, W_self1, W_neigh1, b1.reshape(1, D))
    agg2 = _sc_aggregate_l2()(
        h1.reshape(4 * N_NODES, 64), srcidx, dstidx)
    agg2 = agg2.transpose(1, 0, 2).reshape(N_NODES, D)
    h2 = tc(h1, agg2, deg2d, W_self2, W_neigh2, b2.reshape(1, D))
    return h2


# speedup vs baseline: 6.4388x; 1.0707x over previous
import functools

import jax
import jax.numpy as jnp
from jax import lax
from jax.experimental import pallas as pl
from jax.experimental.pallas import tpu as pltpu
from jax.experimental.pallas import tpu_sc as plsc

N_NODES = 10000
D = 256
E = 160000
N_TILES = 16
EDGES_PER_TILE = E // N_TILES
CHUNK = 80
N_CHUNKS = EDGES_PER_TILE // CHUNK
ROW_CHUNKS = N_NODES // CHUNK
ROW_LOOPS = -(-ROW_CHUNKS // N_TILES)
DEG_PAD = 10240
DEG_CHUNK = 1024
DEG_ZCHUNK = DEG_PAD // N_TILES


def _make_sc_aggregate(with_deg: bool, groups: int):
    width = 128 // groups
    n_pass = 2 * groups
    if with_deg:
        out_type = [jax.ShapeDtypeStruct((n_pass, N_NODES, width), jnp.float32),
                    jax.ShapeDtypeStruct((DEG_PAD,), jnp.float32)]
    else:
        out_type = jax.ShapeDtypeStruct((n_pass, N_NODES, width), jnp.float32)

    mesh = plsc.VectorSubcoreMesh(core_axis_name="c", subcore_axis_name="s")

    scratch_types = [
        pltpu.VMEM((N_CHUNKS, CHUNK), jnp.int32),
        pltpu.VMEM((N_CHUNKS, CHUNK), jnp.int32),
        pltpu.VMEM((CHUNK, width), jnp.float32),
        pltpu.VMEM((CHUNK, width), jnp.float32),
        pltpu.VMEM((CHUNK, width), jnp.float32),
        pltpu.VMEM((CHUNK, width), jnp.float32),
        pltpu.VMEM((CHUNK,), jnp.float32),
        pltpu.VMEM((DEG_CHUNK,), jnp.float32),
        pltpu.VMEM_SHARED((N_NODES, width), jnp.float32),
        pltpu.VMEM_SHARED((DEG_PAD,), jnp.float32),
        pltpu.SemaphoreType.DMA,
        pltpu.SemaphoreType.DMA,
        pltpu.SemaphoreType.DMA,
        pltpu.SemaphoreType.DMA,
        pltpu.SemaphoreType.DMA,
        pltpu.SemaphoreType.DMA,
        pltpu.SemaphoreType.DMA,
        pltpu.SemaphoreType.DMA,
        pltpu.SemaphoreType.DMA,
    ]

    @functools.partial(
        pl.kernel, mesh=mesh, out_type=out_type, scratch_types=scratch_types,
        compiler_params=pltpu.CompilerParams(use_tc_tiling_on_sc=False))
    def sc_aggregate(tbl_hbm, srcidx_hbm, dstidx_hbm, *rest):
        if with_deg:
            agg_hbm, deg_hbm = rest[0], rest[1]
            scr = rest[2:]
        else:
            agg_hbm = rest[0]
            scr = rest[1:]
        (src_v, dst_v, buf0, buf1, buf2, buf3, ones_v, dzero_v, acc, deg_acc,
         gsem0, gsem1, gsem2, gsem3, ssem0, ssem1, ssem2, ssem3, dsem) = scr
        bufs = (buf0, buf1, buf2, buf3)
        gsems = (gsem0, gsem1, gsem2, gsem3)
        ssems = (ssem0, ssem1, ssem2, ssem3)
        c = lax.axis_index("c")
        s = lax.axis_index("s")

        pltpu.sync_copy(srcidx_hbm.at[s], src_v)
        pltpu.sync_copy(dstidx_hbm.at[s], dst_v)

        for k in range(CHUNK // 16):
            ones_v[pl.ds(k * 16, 16)] = jnp.ones((16,), jnp.float32)

        def dz_step(i, carry):
            dzero_v[pl.ds(i * 16, 16)] = jnp.zeros((16,), jnp.float32)
            return carry

        lax.fori_loop(0, DEG_CHUNK // 16, dz_step, 0)

        def add_to_src(delta):
            def addc_step(i, carry):
                for k in range(CHUNK // 16):
                    sl = pl.ds(k * 16, 16)
                    src_v[i, sl] = src_v[i, sl] + delta
                return carry

            lax.fori_loop(0, N_CHUNKS, addc_step, 0)

        def gather(j, buf, sem):
            return pltpu.make_async_copy(tbl_hbm.at[src_v.at[j]], buf, sem)

        for g in range(groups):
            first = g == 0
            add_to_src(groups * c if first else 1)

            def zbuf_step(i, carry):
                for k in range(width // 16):
                    buf0[i, pl.ds(k * 16, 16)] = jnp.zeros((16,), jnp.float32)
                return carry

            lax.fori_loop(0, CHUNK, zbuf_step, 0)

            def zero_step(k, carry):
                i = s + k * N_TILES

                @pl.when(i < ROW_CHUNKS)
                def _():
                    pltpu.sync_copy(buf0, acc.at[pl.ds(i * CHUNK, CHUNK)])
                return carry

            lax.fori_loop(0, ROW_LOOPS, zero_step, 0)
            if with_deg and first:
                @pl.when(c == 0)
                def _():
                    pltpu.sync_copy(
                        dzero_v.at[pl.ds(0, DEG_ZCHUNK)],
                        deg_acc.at[pl.ds(s * DEG_ZCHUNK, DEG_ZCHUNK)])
            plsc.subcore_barrier()

            do_deg = with_deg and first

            def scat_start(j, buf, sem):
                pltpu.async_copy(buf, acc.at[dst_v.at[j]], sem, add=True)

            def scat_wait(buf, sem):
                pltpu.make_async_copy(buf, acc.at[dst_v.at[0]], sem).wait()

            def deg_start(j):
                pltpu.async_copy(ones_v, deg_acc.at[dst_v.at[j]], dsem,
                                 add=True)

            def deg_wait():
                pltpu.make_async_copy(ones_v, deg_acc.at[dst_v.at[0]],
                                      dsem).wait()

            gather(0, bufs[0], gsems[0]).start()
            gather(1, bufs[1], gsems[1]).start()

            def quad(t, carry):
                for b in range(4):
                    j = 4 * t + b
                    gather(j, bufs[b], gsems[b]).wait()
                    scat_start(j, bufs[b], ssems[b])
                    if do_deg:
                        @pl.when(c == 0)
                        def _():
                            deg_start(j)

                            @pl.when(t > 0)
                            def _():
                                deg_wait()
                    b2 = (b + 2) % 4
                    if b < 2:
                        @pl.when(t > 0)
                        def _():
                            scat_wait(bufs[b2], ssems[b2])
                        gather(j + 2, bufs[b2], gsems[b2]).start()
                    elif b == 2:
                        scat_wait(bufs[b2], ssems[b2])
                        gather(j + 2, bufs[b2], gsems[b2]).start()
                    else:
                        scat_wait(bufs[b2], ssems[b2])

                        @pl.when(t < (N_CHUNKS - 1) // 4 - 1)
                        def _():
                            gather(j + 2, bufs[b2], gsems[b2]).start()
                return carry

            lax.fori_loop(0, (N_CHUNKS - 1) // 4, quad, 0)
            gather(N_CHUNKS - 1, bufs[0], gsems[0]).wait()
            scat_start(N_CHUNKS - 1, bufs[0], ssems[0])
            scat_wait(bufs[2], ssems[2])
            scat_wait(bufs[3], ssems[3])
            scat_wait(bufs[0], ssems[0])
            if do_deg:
                @pl.when(c == 0)
                def _():
                    deg_start(N_CHUNKS - 1)
                    for _ in range(5):
                        deg_wait()

            plsc.subcore_barrier()

            def out_step(k, carry):
                i = s + k * N_TILES

                @pl.when(i < ROW_CHUNKS)
                def _():
                    rows = pl.ds(i * CHUNK, CHUNK)
                    pltpu.sync_copy(acc.at[rows],
                                    agg_hbm.at[groups * c + g, rows])
                return carry

            lax.fori_loop(0, ROW_LOOPS, out_step, 0)
            if with_deg and first:
                @pl.when(jnp.logical_and(c == 0, s < DEG_PAD // DEG_CHUNK))
                def _():
                    sl = pl.ds(s * DEG_CHUNK, DEG_CHUNK)
                    pltpu.sync_copy(deg_acc.at[sl], dzero_v)
                    pltpu.sync_copy(dzero_v, deg_hbm.at[sl])

    return sc_aggregate


@functools.lru_cache(maxsize=None)
def _sc_aggregate_l1():
    return _make_sc_aggregate(with_deg=True, groups=2)


@functools.lru_cache(maxsize=None)
def _sc_aggregate_l2():
    return _make_sc_aggregate(with_deg=False, groups=2)


BM = 1000


def _tc_layer_body(h_ref, agg_ref, deg_ref, ws_ref, wn_ref, b_ref, out_ref):
    inv = 1.0 / jnp.maximum(deg_ref[...], 1.0)
    out = (jnp.dot(h_ref[...], ws_ref[...], preferred_element_type=jnp.float32)
           + b_ref[...])
    wn = wn_ref[...]
    for p in range(4):
        out += jnp.dot(agg_ref[p] * inv, wn[64 * p:64 * (p + 1), :],
                       preferred_element_type=jnp.float32)
    out_ref[...] = jnp.maximum(out, 0.0)


@functools.lru_cache(maxsize=None)
def _tc_layer():
    return pl.pallas_call(
        _tc_layer_body,
        grid=(N_NODES // BM,),
        in_specs=[
            pl.BlockSpec((BM, D), lambda i: (i, 0)),
            pl.BlockSpec((4, BM, 64), lambda i: (0, i, 0)),
            pl.BlockSpec((BM, 1), lambda i: (i, 0)),
            pl.BlockSpec((D, D), lambda i: (0, 0)),
            pl.BlockSpec((D, D), lambda i: (0, 0)),
            pl.BlockSpec((1, D), lambda i: (0, 0)),
        ],
        out_specs=pl.BlockSpec((BM, D), lambda i: (i, 0)),
        out_shape=jax.ShapeDtypeStruct((N_NODES, D), jnp.float32),
    )


def kernel(features, edge_index, W_self1, W_neigh1, b1, W_self2, W_neigh2, b2):
    src = edge_index[0].astype(jnp.int32)
    dst = edge_index[1].astype(jnp.int32)
    srcidx = (src * 4).reshape(N_TILES, N_CHUNKS, CHUNK)
    dstidx = dst.reshape(N_TILES, N_CHUNKS, CHUNK)

    tc = _tc_layer()
    agg1, deg = _sc_aggregate_l1()(
        features.reshape(4 * N_NODES, 64), srcidx, dstidx)
    deg2d = deg[:N_NODES].reshape(N_NODES, 1)
    h1 = tc(features, agg1, deg2d, W_self1, W_neigh1, b1.reshape(1, D))
    agg2 = _sc_aggregate_l2()(
        h1.reshape(4 * N_NODES, 64), srcidx, dstidx)
    h2 = tc(h1, agg2, deg2d, W_self2, W_neigh2, b2.reshape(1, D))
    return h2


# speedup vs baseline: 6.4831x; 1.0069x over previous
import functools

import jax
import jax.numpy as jnp
from jax import lax
from jax.experimental import pallas as pl
from jax.experimental.pallas import tpu as pltpu
from jax.experimental.pallas import tpu_sc as plsc

N_NODES = 10000
D = 256
E = 160000
N_TILES = 16
EDGES_PER_TILE = E // N_TILES
CHUNK = 80
N_CHUNKS = EDGES_PER_TILE // CHUNK
ROW_CHUNKS = N_NODES // CHUNK
ROW_LOOPS = -(-ROW_CHUNKS // N_TILES)
DEG_PAD = 10240
DEG_CHUNK = 1024
DEG_ZCHUNK = DEG_PAD // N_TILES


def _make_sc_aggregate(with_deg: bool, groups: int):
    width = 128 // groups
    n_pass = 2 * groups
    if with_deg:
        out_type = [jax.ShapeDtypeStruct((n_pass, N_NODES, width), jnp.float32),
                    jax.ShapeDtypeStruct((DEG_PAD,), jnp.float32)]
    else:
        out_type = jax.ShapeDtypeStruct((n_pass, N_NODES, width), jnp.float32)

    mesh = plsc.VectorSubcoreMesh(core_axis_name="c", subcore_axis_name="s")

    scratch_types = [
        pltpu.VMEM((N_CHUNKS, CHUNK), jnp.int32),
        pltpu.VMEM((N_CHUNKS, CHUNK), jnp.int32),
        pltpu.VMEM((CHUNK, width), jnp.float32),
        pltpu.VMEM((CHUNK, width), jnp.float32),
        pltpu.VMEM((CHUNK, width), jnp.float32),
        pltpu.VMEM((CHUNK, width), jnp.float32),
        pltpu.VMEM((CHUNK,), jnp.float32),
        pltpu.VMEM((DEG_CHUNK,), jnp.float32),
        pltpu.VMEM_SHARED((N_NODES, width), jnp.float32),
        pltpu.VMEM_SHARED((DEG_PAD,), jnp.float32),
        pltpu.SemaphoreType.DMA,
        pltpu.SemaphoreType.DMA,
        pltpu.SemaphoreType.DMA,
        pltpu.SemaphoreType.DMA,
        pltpu.SemaphoreType.DMA,
        pltpu.SemaphoreType.DMA,
        pltpu.SemaphoreType.DMA,
        pltpu.SemaphoreType.DMA,
        pltpu.SemaphoreType.DMA,
    ]

    @functools.partial(
        pl.kernel, mesh=mesh, out_type=out_type, scratch_types=scratch_types,
        compiler_params=pltpu.CompilerParams(use_tc_tiling_on_sc=False))
    def sc_aggregate(tbl_hbm, srcidx_hbm, dstidx_hbm, *rest):
        if with_deg:
            agg_hbm, deg_hbm = rest[0], rest[1]
            scr = rest[2:]
        else:
            agg_hbm = rest[0]
            scr = rest[1:]
        (src_v, dst_v, buf0, buf1, buf2, buf3, ones_v, dzero_v, acc, deg_acc,
         gsem0, gsem1, gsem2, gsem3, ssem0, ssem1, ssem2, ssem3, dsem) = scr
        bufs = (buf0, buf1, buf2, buf3)
        gsems = (gsem0, gsem1, gsem2, gsem3)
        ssems = (ssem0, ssem1, ssem2, ssem3)
        c = lax.axis_index("c")
        s = lax.axis_index("s")

        pltpu.sync_copy(srcidx_hbm.at[s], src_v)
        pltpu.sync_copy(dstidx_hbm.at[s], dst_v)

        for k in range(CHUNK // 16):
            ones_v[pl.ds(k * 16, 16)] = jnp.ones((16,), jnp.float32)

        def dz_step(i, carry):
            dzero_v[pl.ds(i * 16, 16)] = jnp.zeros((16,), jnp.float32)
            return carry

        lax.fori_loop(0, DEG_CHUNK // 16, dz_step, 0)

        def add_to_src(delta):
            def addc_step(i, carry):
                for k in range(CHUNK // 16):
                    sl = pl.ds(k * 16, 16)
                    src_v[i, sl] = src_v[i, sl] + delta
                return carry

            lax.fori_loop(0, N_CHUNKS, addc_step, 0)

        def gather(j, buf, sem):
            return pltpu.make_async_copy(tbl_hbm.at[src_v.at[j]], buf, sem)

        for g in range(groups):
            first = g == 0
            add_to_src(groups * c if first else 1)

            def zbuf_step(i, carry):
                for k in range(width // 16):
                    buf0[i, pl.ds(k * 16, 16)] = jnp.zeros((16,), jnp.float32)
                return carry

            lax.fori_loop(0, CHUNK, zbuf_step, 0)

            def zero_step(k, carry):
                i = s + k * N_TILES

                @pl.when(i < ROW_CHUNKS)
                def _():
                    pltpu.sync_copy(buf0, acc.at[pl.ds(i * CHUNK, CHUNK)])
                return carry

            lax.fori_loop(0, ROW_LOOPS, zero_step, 0)
            if with_deg and first:
                @pl.when(c == 0)
                def _():
                    pltpu.sync_copy(
                        dzero_v.at[pl.ds(0, DEG_ZCHUNK)],
                        deg_acc.at[pl.ds(s * DEG_ZCHUNK, DEG_ZCHUNK)])
            plsc.subcore_barrier()

            do_deg = with_deg and first

            def scat_start(j, buf, sem):
                pltpu.async_copy(buf, acc.at[dst_v.at[j]], sem, add=True)

            def scat_wait(buf, sem):
                pltpu.make_async_copy(buf, acc.at[dst_v.at[0]], sem).wait()

            def deg_start(j):
                pltpu.async_copy(ones_v, deg_acc.at[dst_v.at[j]], dsem,
                                 add=True)

            def deg_wait():
                pltpu.make_async_copy(ones_v, deg_acc.at[dst_v.at[0]],
                                      dsem).wait()

            gather(0, bufs[0], gsems[0]).start()
            gather(1, bufs[1], gsems[1]).start()

            def quad(t, carry):
                for b in range(4):
                    j = 4 * t + b
                    gather(j, bufs[b], gsems[b]).wait()
                    scat_start(j, bufs[b], ssems[b])
                    if do_deg:
                        @pl.when(c == 0)
                        def _():
                            deg_start(j)

                            @pl.when(t > 0)
                            def _():
                                deg_wait()
                    b2 = (b + 2) % 4
                    if b < 2:
                        @pl.when(t > 0)
                        def _():
                            scat_wait(bufs[b2], ssems[b2])
                        gather(j + 2, bufs[b2], gsems[b2]).start()
                    elif b == 2:
                        scat_wait(bufs[b2], ssems[b2])
                        gather(j + 2, bufs[b2], gsems[b2]).start()
                    else:
                        scat_wait(bufs[b2], ssems[b2])

                        @pl.when(t < (N_CHUNKS - 1) // 4 - 1)
                        def _():
                            gather(j + 2, bufs[b2], gsems[b2]).start()
                return carry

            lax.fori_loop(0, (N_CHUNKS - 1) // 4, quad, 0)
            gather(N_CHUNKS - 1, bufs[0], gsems[0]).wait()
            scat_start(N_CHUNKS - 1, bufs[0], ssems[0])
            scat_wait(bufs[2], ssems[2])
            scat_wait(bufs[3], ssems[3])
            scat_wait(bufs[0], ssems[0])
            if do_deg:
                @pl.when(c == 0)
                def _():
                    deg_start(N_CHUNKS - 1)
                    for _ in range(5):
                        deg_wait()

            plsc.subcore_barrier()

            def out_step(k, carry):
                i = s + k * N_TILES

                @pl.when(i < ROW_CHUNKS)
                def _():
                    rows = pl.ds(i * CHUNK, CHUNK)
                    pltpu.sync_copy(acc.at[rows],
                                    agg_hbm.at[groups * c + g, rows])
                return carry

            lax.fori_loop(0, ROW_LOOPS, out_step, 0)
            if with_deg and first:
                @pl.when(jnp.logical_and(c == 0, s < DEG_PAD // DEG_CHUNK))
                def _():
                    sl = pl.ds(s * DEG_CHUNK, DEG_CHUNK)
                    pltpu.sync_copy(deg_acc.at[sl], dzero_v)
                    pltpu.sync_copy(dzero_v, deg_hbm.at[sl])

    return sc_aggregate


@functools.lru_cache(maxsize=None)
def _sc_aggregate_l1():
    return _make_sc_aggregate(with_deg=True, groups=2)


@functools.lru_cache(maxsize=None)
def _sc_aggregate_l2():
    return _make_sc_aggregate(with_deg=False, groups=2)


BM = 1000


def _tc_self_body(h_ref, ws_ref, b_ref, out_ref):
    out_ref[...] = (
        jnp.dot(h_ref[...], ws_ref[...], preferred_element_type=jnp.float32)
        + b_ref[...])


@functools.lru_cache(maxsize=None)
def _tc_self():
    return pl.pallas_call(
        _tc_self_body,
        grid=(N_NODES // BM,),
        in_specs=[
            pl.BlockSpec((BM, D), lambda i: (i, 0)),
            pl.BlockSpec((D, D), lambda i: (0, 0)),
            pl.BlockSpec((1, D), lambda i: (0, 0)),
        ],
        out_specs=pl.BlockSpec((BM, D), lambda i: (i, 0)),
        out_shape=jax.ShapeDtypeStruct((N_NODES, D), jnp.float32),
    )


def _tc_combine_body(s_ref, agg_ref, deg_ref, wn_ref, out_ref):
    inv = 1.0 / jnp.maximum(deg_ref[...], 1.0)
    out = s_ref[...]
    wn = wn_ref[...]
    for p in range(4):
        out += jnp.dot(agg_ref[p] * inv, wn[64 * p:64 * (p + 1), :],
                       preferred_element_type=jnp.float32)
    out_ref[...] = jnp.maximum(out, 0.0)


@functools.lru_cache(maxsize=None)
def _tc_combine():
    return pl.pallas_call(
        _tc_combine_body,
        grid=(N_NODES // BM,),
        in_specs=[
            pl.BlockSpec((BM, D), lambda i: (i, 0)),
            pl.BlockSpec((4, BM, 64), lambda i: (0, i, 0)),
            pl.BlockSpec((BM, 1), lambda i: (i, 0)),
            pl.BlockSpec((D, D), lambda i: (0, 0)),
        ],
        out_specs=pl.BlockSpec((BM, D), lambda i: (i, 0)),
        out_shape=jax.ShapeDtypeStruct((N_NODES, D), jnp.float32),
    )


def kernel(features, edge_index, W_self1, W_neigh1, b1, W_self2, W_neigh2, b2):
    src = edge_index[0].astype(jnp.int32)
    dst = edge_index[1].astype(jnp.int32)
    srcidx = (src * 4).reshape(N_TILES, N_CHUNKS, CHUNK)
    dstidx = dst.reshape(N_TILES, N_CHUNKS, CHUNK)

    tc_self = _tc_self()
    tc_combine = _tc_combine()
    agg1, deg = _sc_aggregate_l1()(
        features.reshape(4 * N_NODES, 64), srcidx, dstidx)
    s1 = tc_self(features, W_self1, b1.reshape(1, D))
    deg2d = deg[:N_NODES].reshape(N_NODES, 1)
    h1 = tc_combine(s1, agg1, deg2d, W_neigh1)
    agg2 = _sc_aggregate_l2()(
        h1.reshape(4 * N_NODES, 64), srcidx, dstidx)
    s2 = tc_self(h1, W_self2, b2.reshape(1, D))
    h2 = tc_combine(s2, agg2, deg2d, W_neigh2)
    return h2


# speedup vs baseline: 7.4780x; 1.1535x over previous
import functools

import jax
import jax.numpy as jnp
from jax import lax
from jax.experimental import pallas as pl
from jax.experimental.pallas import tpu as pltpu
from jax.experimental.pallas import tpu_sc as plsc

N_NODES = 10000
D = 256
E = 160000
N_TILES = 16
EDGES_PER_TILE = E // N_TILES
CHUNK = 80
N_CHUNKS = EDGES_PER_TILE // CHUNK
ROW_CHUNKS = N_NODES // CHUNK
ROW_LOOPS = -(-ROW_CHUNKS // N_TILES)
DEG_PAD = 10240
DEG_CHUNK = 1024
DEG_ZCHUNK = DEG_PAD // N_TILES
NBUF = 8
AHEAD = 4
N_LOOP = (N_CHUNKS // NBUF) * NBUF


def _make_sc_aggregate(with_deg: bool, groups: int):
    width = 128 // groups
    n_pass = 2 * groups
    if with_deg:
        out_type = [jax.ShapeDtypeStruct((n_pass, N_NODES, width), jnp.float32),
                    jax.ShapeDtypeStruct((DEG_PAD,), jnp.float32)]
    else:
        out_type = jax.ShapeDtypeStruct((n_pass, N_NODES, width), jnp.float32)

    mesh = plsc.VectorSubcoreMesh(core_axis_name="c", subcore_axis_name="s")

    scratch_types = [
        pltpu.VMEM((N_CHUNKS, CHUNK), jnp.int32),
        pltpu.VMEM((N_CHUNKS, CHUNK), jnp.int32),
        [pltpu.VMEM((CHUNK, width), jnp.float32)] * NBUF,
        pltpu.VMEM((CHUNK,), jnp.float32),
        pltpu.VMEM((DEG_CHUNK,), jnp.float32),
        pltpu.VMEM_SHARED((N_NODES, width), jnp.float32),
        pltpu.VMEM_SHARED((DEG_PAD,), jnp.float32),
        [pltpu.SemaphoreType.DMA] * NBUF,
        [pltpu.SemaphoreType.DMA] * NBUF,
        pltpu.SemaphoreType.DMA,
    ]

    @functools.partial(
        pl.kernel, mesh=mesh, out_type=out_type, scratch_types=scratch_types,
        compiler_params=pltpu.CompilerParams(use_tc_tiling_on_sc=False))
    def sc_aggregate(tbl_hbm, srcidx_hbm, dstidx_hbm, *rest):
        if with_deg:
            agg_hbm, deg_hbm = rest[0], rest[1]
            scr = rest[2:]
        else:
            agg_hbm = rest[0]
            scr = rest[1:]
        (src_v, dst_v, bufs, ones_v, dzero_v, acc, deg_acc,
         gsems, ssems, dsem) = scr
        c = lax.axis_index("c")
        s = lax.axis_index("s")

        pltpu.sync_copy(srcidx_hbm.at[s], src_v)
        pltpu.sync_copy(dstidx_hbm.at[s], dst_v)

        for k in range(CHUNK // 16):
            ones_v[pl.ds(k * 16, 16)] = jnp.ones((16,), jnp.float32)

        def dz_step(i, carry):
            dzero_v[pl.ds(i * 16, 16)] = jnp.zeros((16,), jnp.float32)
            return carry

        lax.fori_loop(0, DEG_CHUNK // 16, dz_step, 0)

        def add_to_src(delta):
            def addc_step(i, carry):
                for k in range(CHUNK // 16):
                    sl = pl.ds(k * 16, 16)
                    src_v[i, sl] = src_v[i, sl] + delta
                return carry

            lax.fori_loop(0, N_CHUNKS, addc_step, 0)

        def gather(j, buf, sem):
            return pltpu.make_async_copy(tbl_hbm.at[src_v.at[j]], buf, sem)

        for g in range(groups):
            first = g == 0
            add_to_src(groups * c if first else 1)

            def zbuf_step(i, carry):
                for k in range(width // 16):
                    bufs[0][i, pl.ds(k * 16, 16)] = jnp.zeros((16,), jnp.float32)
                return carry

            lax.fori_loop(0, CHUNK, zbuf_step, 0)

            def zero_step(k, carry):
                i = s + k * N_TILES

                @pl.when(i < ROW_CHUNKS)
                def _():
                    pltpu.sync_copy(bufs[0], acc.at[pl.ds(i * CHUNK, CHUNK)])
                return carry

            lax.fori_loop(0, ROW_LOOPS, zero_step, 0)
            if with_deg and first:
                @pl.when(c == 0)
                def _():
                    pltpu.sync_copy(
                        dzero_v.at[pl.ds(0, DEG_ZCHUNK)],
                        deg_acc.at[pl.ds(s * DEG_ZCHUNK, DEG_ZCHUNK)])
            plsc.subcore_barrier()

            do_deg = with_deg and first

            def scat_start(j, buf, sem):
                pltpu.async_copy(buf, acc.at[dst_v.at[j]], sem, add=True)

            def scat_wait(buf, sem):
                pltpu.make_async_copy(buf, acc.at[dst_v.at[0]], sem).wait()

            def deg_start(j):
                pltpu.async_copy(ones_v, deg_acc.at[dst_v.at[j]], dsem,
                                 add=True)

            def deg_wait():
                pltpu.make_async_copy(ones_v, deg_acc.at[dst_v.at[0]],
                                      dsem).wait()

            for b in range(AHEAD):
                gather(b, bufs[b], gsems[b]).start()

            def octet(t, carry):
                for b in range(NBUF):
                    j = NBUF * t + b
                    b2 = (b + AHEAD) % NBUF
                    gather(j, bufs[b], gsems[b]).wait()
                    scat_start(j, bufs[b], ssems[b])
                    if do_deg:
                        @pl.when(c == 0)
                        def _():
                            deg_start(j)

                            @pl.when(t > 0)
                            def _():
                                deg_wait()
                    if b < AHEAD:
                        @pl.when(t > 0)
                        def _():
                            scat_wait(bufs[b2], ssems[b2])
                        gather(j + AHEAD, bufs[b2], gsems[b2]).start()
                    else:
                        scat_wait(bufs[b2], ssems[b2])
                        gather(j + AHEAD, bufs[b2], gsems[b2]).start()
                return carry

            lax.fori_loop(0, N_LOOP // NBUF, octet, 0)
            for j in range(N_LOOP, N_CHUNKS):
                b = j % NBUF
                b2 = (b + AHEAD) % NBUF
                gather(j, bufs[b], gsems[b]).wait()
                scat_start(j, bufs[b], ssems[b])
                scat_wait(bufs[b2], ssems[b2])
                if j + AHEAD < N_CHUNKS:
                    gather(j + AHEAD, bufs[b2], gsems[b2]).start()
                if do_deg:
                    @pl.when(c == 0)
                    def _():
                        deg_start(j)
                        deg_wait()
            for j in range(N_CHUNKS - AHEAD, N_CHUNKS):
                b = j % NBUF
                scat_wait(bufs[b], ssems[b])
            if do_deg:
                @pl.when(c == 0)
                def _():
                    for _ in range(NBUF):
                        deg_wait()

            plsc.subcore_barrier()

            def out_step(k, carry):
                i = s + k * N_TILES

                @pl.when(i < ROW_CHUNKS)
                def _():
                    rows = pl.ds(i * CHUNK, CHUNK)
                    pltpu.sync_copy(acc.at[rows],
                                    agg_hbm.at[groups * c + g, rows])
                return carry

            lax.fori_loop(0, ROW_LOOPS, out_step, 0)
            if with_deg and first:
                @pl.when(jnp.logical_and(c == 0, s < DEG_PAD // DEG_CHUNK))
                def _():
                    sl = pl.ds(s * DEG_CHUNK, DEG_CHUNK)
                    pltpu.sync_copy(deg_acc.at[sl], dzero_v)
                    pltpu.sync_copy(dzero_v, deg_hbm.at[sl])

    return sc_aggregate


@functools.lru_cache(maxsize=None)
def _sc_aggregate_l1():
    return _make_sc_aggregate(with_deg=True, groups=2)


@functools.lru_cache(maxsize=None)
def _sc_aggregate_l2():
    return _make_sc_aggregate(with_deg=False, groups=2)


BM = 1000


def _tc_self_body(h_ref, ws_ref, b_ref, out_ref):
    out_ref[...] = (
        jnp.dot(h_ref[...], ws_ref[...], preferred_element_type=jnp.float32)
        + b_ref[...])


@functools.lru_cache(maxsize=None)
def _tc_self():
    return pl.pallas_call(
        _tc_self_body,
        grid=(N_NODES // BM,),
        in_specs=[
            pl.BlockSpec((BM, D), lambda i: (i, 0)),
            pl.BlockSpec((D, D), lambda i: (0, 0)),
            pl.BlockSpec((1, D), lambda i: (0, 0)),
        ],
        out_specs=pl.BlockSpec((BM, D), lambda i: (i, 0)),
        out_shape=jax.ShapeDtypeStruct((N_NODES, D), jnp.float32),
    )


def _tc_combine_body(s_ref, agg_ref, deg_ref, wn_ref, out_ref):
    inv = 1.0 / jnp.maximum(deg_ref[...], 1.0)
    out = s_ref[...]
    wn = wn_ref[...]
    for p in range(4):
        out += jnp.dot(agg_ref[p] * inv, wn[64 * p:64 * (p + 1), :],
                       preferred_element_type=jnp.float32)
    out_ref[...] = jnp.maximum(out, 0.0)


@functools.lru_cache(maxsize=None)
def _tc_combine():
    return pl.pallas_call(
        _tc_combine_body,
        grid=(N_NODES // BM,),
        in_specs=[
            pl.BlockSpec((BM, D), lambda i: (i, 0)),
            pl.BlockSpec((4, BM, 64), lambda i: (0, i, 0)),
            pl.BlockSpec((BM, 1), lambda i: (i, 0)),
            pl.BlockSpec((D, D), lambda i: (0, 0)),
        ],
        out_specs=pl.BlockSpec((BM, D), lambda i: (i, 0)),
        out_shape=jax.ShapeDtypeStruct((N_NODES, D), jnp.float32),
    )


def kernel(features, edge_index, W_self1, W_neigh1, b1, W_self2, W_neigh2, b2):
    src = edge_index[0].astype(jnp.int32)
    dst = edge_index[1].astype(jnp.int32)
    srcidx = (src * 4).reshape(N_TILES, N_CHUNKS, CHUNK)
    dstidx = dst.reshape(N_TILES, N_CHUNKS, CHUNK)

    tc_self = _tc_self()
    tc_combine = _tc_combine()
    agg1, deg = _sc_aggregate_l1()(
        features.reshape(4 * N_NODES, 64), srcidx, dstidx)
    s1 = tc_self(features, W_self1, b1.reshape(1, D))
    deg2d = deg[:N_NODES].reshape(N_NODES, 1)
    h1 = tc_combine(s1, agg1, deg2d, W_neigh1)
    agg2 = _sc_aggregate_l2()(
        h1.reshape(4 * N_NODES, 64), srcidx, dstidx)
    s2 = tc_self(h1, W_self2, b2.reshape(1, D))
    h2 = tc_combine(s2, agg2, deg2d, W_neigh2)
    return h2


# speedup vs baseline: 7.6751x; 1.0264x over previous
import functools

import jax
import jax.numpy as jnp
from jax import lax
from jax.experimental import pallas as pl
from jax.experimental.pallas import tpu as pltpu
from jax.experimental.pallas import tpu_sc as plsc

N_NODES = 10000
D = 256
E = 160000
N_TILES = 16
EDGES_PER_TILE = E // N_TILES
CHUNK = 80
N_CHUNKS = EDGES_PER_TILE // CHUNK
ROW_CHUNKS = N_NODES // CHUNK
ROW_LOOPS = -(-ROW_CHUNKS // N_TILES)
DEG_PAD = 10240
DEG_CHUNK = 1024
DEG_ZCHUNK = DEG_PAD // N_TILES
NBUF = 12
AHEAD = 6
N_LOOP = (N_CHUNKS // NBUF) * NBUF


def _make_sc_aggregate(with_deg: bool, groups: int):
    width = 128 // groups
    n_pass = 2 * groups
    if with_deg:
        out_type = [jax.ShapeDtypeStruct((n_pass, N_NODES, width), jnp.float32),
                    jax.ShapeDtypeStruct((DEG_PAD,), jnp.float32)]
    else:
        out_type = jax.ShapeDtypeStruct((n_pass, N_NODES, width), jnp.float32)

    mesh = plsc.VectorSubcoreMesh(core_axis_name="c", subcore_axis_name="s")

    scratch_types = [
        pltpu.VMEM((N_CHUNKS, CHUNK), jnp.int32),
        pltpu.VMEM((N_CHUNKS, CHUNK), jnp.int32),
        [pltpu.VMEM((CHUNK, width), jnp.float32)] * NBUF,
        pltpu.VMEM((CHUNK,), jnp.float32),
        pltpu.VMEM((DEG_CHUNK,), jnp.float32),
        pltpu.VMEM_SHARED((N_NODES, width), jnp.float32),
        pltpu.VMEM_SHARED((DEG_PAD,), jnp.float32),
        [pltpu.SemaphoreType.DMA] * NBUF,
        [pltpu.SemaphoreType.DMA] * NBUF,
        pltpu.SemaphoreType.DMA,
    ]

    @functools.partial(
        pl.kernel, mesh=mesh, out_type=out_type, scratch_types=scratch_types,
        compiler_params=pltpu.CompilerParams(use_tc_tiling_on_sc=False))
    def sc_aggregate(tbl_hbm, srcidx_hbm, dstidx_hbm, *rest):
        if with_deg:
            agg_hbm, deg_hbm = rest[0], rest[1]
            scr = rest[2:]
        else:
            agg_hbm = rest[0]
            scr = rest[1:]
        (src_v, dst_v, bufs, ones_v, dzero_v, acc, deg_acc,
         gsems, ssems, dsem) = scr
        c = lax.axis_index("c")
        s = lax.axis_index("s")

        pltpu.sync_copy(srcidx_hbm.at[s], src_v)
        pltpu.sync_copy(dstidx_hbm.at[s], dst_v)

        for k in range(CHUNK // 16):
            ones_v[pl.ds(k * 16, 16)] = jnp.ones((16,), jnp.float32)

        def dz_step(i, carry):
            dzero_v[pl.ds(i * 16, 16)] = jnp.zeros((16,), jnp.float32)
            return carry

        lax.fori_loop(0, DEG_CHUNK // 16, dz_step, 0)

        def add_to_src(delta):
            def addc_step(i, carry):
                for k in range(CHUNK // 16):
                    sl = pl.ds(k * 16, 16)
                    src_v[i, sl] = src_v[i, sl] + delta
                return carry

            lax.fori_loop(0, N_CHUNKS, addc_step, 0)

        def gather(j, buf, sem):
            return pltpu.make_async_copy(tbl_hbm.at[src_v.at[j]], buf, sem)

        for g in range(groups):
            first = g == 0
            add_to_src(groups * c if first else 1)

            def zbuf_step(i, carry):
                for k in range(width // 16):
                    bufs[0][i, pl.ds(k * 16, 16)] = jnp.zeros((16,), jnp.float32)
                return carry

            lax.fori_loop(0, CHUNK, zbuf_step, 0)

            def zero_step(k, carry):
                i = s + k * N_TILES

                @pl.when(i < ROW_CHUNKS)
                def _():
                    pltpu.sync_copy(bufs[0], acc.at[pl.ds(i * CHUNK, CHUNK)])
                return carry

            lax.fori_loop(0, ROW_LOOPS, zero_step, 0)
            if with_deg and first:
                @pl.when(c == 0)
                def _():
                    pltpu.sync_copy(
                        dzero_v.at[pl.ds(0, DEG_ZCHUNK)],
                        deg_acc.at[pl.ds(s * DEG_ZCHUNK, DEG_ZCHUNK)])
            plsc.subcore_barrier()

            do_deg = with_deg and first

            def scat_start(j, buf, sem):
                pltpu.async_copy(buf, acc.at[dst_v.at[j]], sem, add=True)

            def scat_wait(buf, sem):
                pltpu.make_async_copy(buf, acc.at[dst_v.at[0]], sem).wait()

            def deg_start(j):
                pltpu.async_copy(ones_v, deg_acc.at[dst_v.at[j]], dsem,
                                 add=True)

            def deg_wait():
                pltpu.make_async_copy(ones_v, deg_acc.at[dst_v.at[0]],
                                      dsem).wait()

            for b in range(AHEAD):
                gather(b, bufs[b], gsems[b]).start()

            def octet(t, carry):
                for b in range(NBUF):
                    j = NBUF * t + b
                    b2 = (b + AHEAD) % NBUF
                    gather(j, bufs[b], gsems[b]).wait()
                    scat_start(j, bufs[b], ssems[b])
                    if do_deg:
                        @pl.when(c == 0)
                        def _():
                            deg_start(j)

                            @pl.when(t > 0)
                            def _():
                                deg_wait()
                    if b < AHEAD:
                        @pl.when(t > 0)
                        def _():
                            scat_wait(bufs[b2], ssems[b2])
                        gather(j + AHEAD, bufs[b2], gsems[b2]).start()
                    else:
                        scat_wait(bufs[b2], ssems[b2])
                        gather(j + AHEAD, bufs[b2], gsems[b2]).start()
                return carry

            lax.fori_loop(0, N_LOOP // NBUF, octet, 0)
            for j in range(N_LOOP, N_CHUNKS):
                b = j % NBUF
                b2 = (b + AHEAD) % NBUF
                gather(j, bufs[b], gsems[b]).wait()
                scat_start(j, bufs[b], ssems[b])
                scat_wait(bufs[b2], ssems[b2])
                if j + AHEAD < N_CHUNKS:
                    gather(j + AHEAD, bufs[b2], gsems[b2]).start()
                if do_deg:
                    @pl.when(c == 0)
                    def _():
                        deg_start(j)
                        deg_wait()
            for j in range(N_CHUNKS - AHEAD, N_CHUNKS):
                b = j % NBUF
                scat_wait(bufs[b], ssems[b])
            if do_deg:
                @pl.when(c == 0)
                def _():
                    for _ in range(NBUF):
                        deg_wait()

            plsc.subcore_barrier()

            def out_step(k, carry):
                i = s + k * N_TILES

                @pl.when(i < ROW_CHUNKS)
                def _():
                    rows = pl.ds(i * CHUNK, CHUNK)
                    pltpu.sync_copy(acc.at[rows],
                                    agg_hbm.at[groups * c + g, rows])
                return carry

            lax.fori_loop(0, ROW_LOOPS, out_step, 0)
            if with_deg and first:
                @pl.when(jnp.logical_and(c == 0, s < DEG_PAD // DEG_CHUNK))
                def _():
                    sl = pl.ds(s * DEG_CHUNK, DEG_CHUNK)
                    pltpu.sync_copy(deg_acc.at[sl], dzero_v)
                    pltpu.sync_copy(dzero_v, deg_hbm.at[sl])

    return sc_aggregate


@functools.lru_cache(maxsize=None)
def _sc_aggregate_l1():
    return _make_sc_aggregate(with_deg=True, groups=2)


@functools.lru_cache(maxsize=None)
def _sc_aggregate_l2():
    return _make_sc_aggregate(with_deg=False, groups=2)


BM = 1000


def _tc_self_body(h_ref, ws_ref, b_ref, out_ref):
    out_ref[...] = (
        jnp.dot(h_ref[...], ws_ref[...], preferred_element_type=jnp.float32)
        + b_ref[...])


@functools.lru_cache(maxsize=None)
def _tc_self():
    return pl.pallas_call(
        _tc_self_body,
        grid=(N_NODES // BM,),
        in_specs=[
            pl.BlockSpec((BM, D), lambda i: (i, 0)),
            pl.BlockSpec((D, D), lambda i: (0, 0)),
            pl.BlockSpec((1, D), lambda i: (0, 0)),
        ],
        out_specs=pl.BlockSpec((BM, D), lambda i: (i, 0)),
        out_shape=jax.ShapeDtypeStruct((N_NODES, D), jnp.float32),
    )


def _tc_combine_body(s_ref, agg_ref, deg_ref, wn_ref, out_ref):
    inv = 1.0 / jnp.maximum(deg_ref[...], 1.0)
    out = s_ref[...]
    wn = wn_ref[...]
    for p in range(4):
        out += jnp.dot(agg_ref[p] * inv, wn[64 * p:64 * (p + 1), :],
                       preferred_element_type=jnp.float32)
    out_ref[...] = jnp.maximum(out, 0.0)


@functools.lru_cache(maxsize=None)
def _tc_combine():
    return pl.pallas_call(
        _tc_combine_body,
        grid=(N_NODES // BM,),
        in_specs=[
            pl.BlockSpec((BM, D), lambda i: (i, 0)),
            pl.BlockSpec((4, BM, 64), lambda i: (0, i, 0)),
            pl.BlockSpec((BM, 1), lambda i: (i, 0)),
            pl.BlockSpec((D, D), lambda i: (0, 0)),
        ],
        out_specs=pl.BlockSpec((BM, D), lambda i: (i, 0)),
        out_shape=jax.ShapeDtypeStruct((N_NODES, D), jnp.float32),
    )


def kernel(features, edge_index, W_self1, W_neigh1, b1, W_self2, W_neigh2, b2):
    src = edge_index[0].astype(jnp.int32)
    dst = edge_index[1].astype(jnp.int32)
    srcidx = (src * 4).reshape(N_TILES, N_CHUNKS, CHUNK)
    dstidx = dst.reshape(N_TILES, N_CHUNKS, CHUNK)

    tc_self = _tc_self()
    tc_combine = _tc_combine()
    agg1, deg = _sc_aggregate_l1()(
        features.reshape(4 * N_NODES, 64), srcidx, dstidx)
    s1 = tc_self(features, W_self1, b1.reshape(1, D))
    deg2d = deg[:N_NODES].reshape(N_NODES, 1)
    h1 = tc_combine(s1, agg1, deg2d, W_neigh1)
    agg2 = _sc_aggregate_l2()(
        h1.reshape(4 * N_NODES, 64), srcidx, dstidx)
    s2 = tc_self(h1, W_self2, b2.reshape(1, D))
    h2 = tc_combine(s2, agg2, deg2d, W_neigh2)
    return h2


# speedup vs baseline: 7.7823x; 1.0140x over previous
import functools

import jax
import jax.numpy as jnp
from jax import lax
from jax.experimental import pallas as pl
from jax.experimental.pallas import tpu as pltpu
from jax.experimental.pallas import tpu_sc as plsc

N_NODES = 10000
D = 256
E = 160000
N_TILES = 16
EDGES_PER_TILE = E // N_TILES
CHUNK = 80
N_CHUNKS = EDGES_PER_TILE // CHUNK
ROW_CHUNKS = N_NODES // CHUNK
ROW_LOOPS = -(-ROW_CHUNKS // N_TILES)
DEG_PAD = 10240
DEG_CHUNK = 1024
DEG_ZCHUNK = DEG_PAD // N_TILES
NBUF = 12
AHEAD = 6
N_LOOP = (N_CHUNKS // NBUF) * NBUF


def _make_sc_aggregate(with_deg: bool, groups: int):
    width = 128 // groups
    n_pass = 2 * groups
    if with_deg:
        out_type = [jax.ShapeDtypeStruct((n_pass, N_NODES, width), jnp.float32),
                    jax.ShapeDtypeStruct((DEG_PAD,), jnp.float32)]
    else:
        out_type = jax.ShapeDtypeStruct((n_pass, N_NODES, width), jnp.float32)

    mesh = plsc.VectorSubcoreMesh(core_axis_name="c", subcore_axis_name="s")

    scratch_types = [
        pltpu.VMEM((N_CHUNKS, CHUNK), jnp.int32),
        pltpu.VMEM((N_CHUNKS, CHUNK), jnp.int32),
        [pltpu.VMEM((CHUNK, width), jnp.float32)] * NBUF,
        pltpu.VMEM((CHUNK,), jnp.float32),
        pltpu.VMEM((DEG_CHUNK,), jnp.float32),
        pltpu.VMEM_SHARED((N_NODES, width), jnp.float32),
        pltpu.VMEM_SHARED((DEG_PAD,), jnp.float32),
        [pltpu.SemaphoreType.DMA] * NBUF,
        [pltpu.SemaphoreType.DMA] * NBUF,
        pltpu.SemaphoreType.DMA,
    ]

    @functools.partial(
        pl.kernel, mesh=mesh, out_type=out_type, scratch_types=scratch_types,
        compiler_params=pltpu.CompilerParams(use_tc_tiling_on_sc=False))
    def sc_aggregate(tbl_hbm, srcidx_hbm, dstidx_hbm, *rest):
        if with_deg:
            agg_hbm, deg_hbm = rest[0], rest[1]
            scr = rest[2:]
        else:
            agg_hbm = rest[0]
            scr = rest[1:]
        (src_v, dst_v, bufs, ones_v, dzero_v, acc, deg_acc,
         gsems, ssems, dsem) = scr
        c = lax.axis_index("c")
        s = lax.axis_index("s")

        pltpu.sync_copy(srcidx_hbm.at[s], src_v)
        pltpu.sync_copy(dstidx_hbm.at[s], dst_v)

        for k in range(CHUNK // 16):
            ones_v[pl.ds(k * 16, 16)] = jnp.ones((16,), jnp.float32)

        def dz_step(i, carry):
            dzero_v[pl.ds(i * 16, 16)] = jnp.zeros((16,), jnp.float32)
            return carry

        lax.fori_loop(0, DEG_CHUNK // 16, dz_step, 0)

        def add_to_src(delta):
            def addc_step(i, carry):
                for k in range(CHUNK // 16):
                    sl = pl.ds(k * 16, 16)
                    src_v[i, sl] = src_v[i, sl] + delta
                return carry

            lax.fori_loop(0, N_CHUNKS, addc_step, 0)

        def gather(j, buf, sem):
            return pltpu.make_async_copy(tbl_hbm.at[src_v.at[j]], buf, sem)

        for g in range(groups):
            first = g == 0
            add_to_src(groups * c if first else 1)

            def zbuf_step(i, carry):
                for k in range(width // 16):
                    bufs[0][i, pl.ds(k * 16, 16)] = jnp.zeros((16,), jnp.float32)
                return carry

            lax.fori_loop(0, CHUNK, zbuf_step, 0)

            def zero_step(k, carry):
                i = s + k * N_TILES

                @pl.when(i < ROW_CHUNKS)
                def _():
                    pltpu.sync_copy(bufs[0], acc.at[pl.ds(i * CHUNK, CHUNK)])
                return carry

            lax.fori_loop(0, ROW_LOOPS, zero_step, 0)
            if with_deg and first:
                @pl.when(c == 0)
                def _():
                    pltpu.sync_copy(
                        dzero_v.at[pl.ds(0, DEG_ZCHUNK)],
                        deg_acc.at[pl.ds(s * DEG_ZCHUNK, DEG_ZCHUNK)])
            plsc.subcore_barrier()

            do_deg = with_deg and first

            def scat_start(j, buf, sem):
                pltpu.async_copy(buf, acc.at[dst_v.at[j]], sem, add=True)

            def scat_wait(buf, sem):
                pltpu.make_async_copy(buf, acc.at[dst_v.at[0]], sem).wait()

            def deg_start(j):
                pltpu.async_copy(ones_v, deg_acc.at[dst_v.at[j]], dsem,
                                 add=True)

            def deg_wait():
                pltpu.make_async_copy(ones_v, deg_acc.at[dst_v.at[0]],
                                      dsem).wait()

            for b in range(AHEAD):
                gather(b, bufs[b], gsems[b]).start()

            def octet(t, carry):
                for b in range(NBUF):
                    j = NBUF * t + b
                    b2 = (b + AHEAD) % NBUF
                    gather(j, bufs[b], gsems[b]).wait()
                    scat_start(j, bufs[b], ssems[b])
                    if do_deg:
                        @pl.when(c == 0)
                        def _():
                            deg_start(j)

                            @pl.when(t > 0)
                            def _():
                                deg_wait()
                    guard = b >= N_CHUNKS - AHEAD - N_LOOP + NBUF

                    def _free_and_refill():
                        if b < AHEAD:
                            @pl.when(t > 0)
                            def _():
                                scat_wait(bufs[b2], ssems[b2])
                        else:
                            scat_wait(bufs[b2], ssems[b2])
                        if guard:
                            @pl.when(t < N_LOOP // NBUF - 1)
                            def _():
                                gather(j + AHEAD, bufs[b2], gsems[b2]).start()
                        else:
                            gather(j + AHEAD, bufs[b2], gsems[b2]).start()

                    _free_and_refill()
                return carry

            lax.fori_loop(0, N_LOOP // NBUF, octet, 0)
            for j in range(N_LOOP, N_CHUNKS):
                b = j % NBUF
                b2 = (b + AHEAD) % NBUF
                gather(j, bufs[b], gsems[b]).wait()
                scat_start(j, bufs[b], ssems[b])
                scat_wait(bufs[b2], ssems[b2])
                if j + AHEAD < N_CHUNKS:
                    gather(j + AHEAD, bufs[b2], gsems[b2]).start()
                if do_deg:
                    @pl.when(c == 0)
                    def _():
                        deg_start(j)
                        deg_wait()
            for j in range(N_CHUNKS - AHEAD, N_CHUNKS):
                b = j % NBUF
                scat_wait(bufs[b], ssems[b])
            if do_deg:
                @pl.when(c == 0)
                def _():
                    for _ in range(NBUF):
                        deg_wait()

            plsc.subcore_barrier()

            def out_step(k, carry):
                i = s + k * N_TILES

                @pl.when(i < ROW_CHUNKS)
                def _():
                    rows = pl.ds(i * CHUNK, CHUNK)
                    pltpu.sync_copy(acc.at[rows],
                                    agg_hbm.at[groups * c + g, rows])
                return carry

            lax.fori_loop(0, ROW_LOOPS, out_step, 0)
            if with_deg and first:
                @pl.when(jnp.logical_and(c == 0, s < DEG_PAD // DEG_CHUNK))
                def _():
                    sl = pl.ds(s * DEG_CHUNK, DEG_CHUNK)
                    pltpu.sync_copy(deg_acc.at[sl], dzero_v)
                    pltpu.sync_copy(dzero_v, deg_hbm.at[sl])

    return sc_aggregate


@functools.lru_cache(maxsize=None)
def _sc_aggregate_l1():
    return _make_sc_aggregate(with_deg=True, groups=2)


@functools.lru_cache(maxsize=None)
def _sc_aggregate_l2():
    return _make_sc_aggregate(with_deg=False, groups=2)


BM = 1000


def _tc_self_body(h_ref, ws_ref, b_ref, out_ref):
    out_ref[...] = (
        jnp.dot(h_ref[...], ws_ref[...], preferred_element_type=jnp.float32)
        + b_ref[...])


@functools.lru_cache(maxsize=None)
def _tc_self():
    return pl.pallas_call(
        _tc_self_body,
        grid=(N_NODES // BM,),
        in_specs=[
            pl.BlockSpec((BM, D), lambda i: (i, 0)),
            pl.BlockSpec((D, D), lambda i: (0, 0)),
            pl.BlockSpec((1, D), lambda i: (0, 0)),
        ],
        out_specs=pl.BlockSpec((BM, D), lambda i: (i, 0)),
        out_shape=jax.ShapeDtypeStruct((N_NODES, D), jnp.float32),
    )


def _tc_combine_body(s_ref, agg_ref, deg_ref, wn_ref, out_ref):
    inv = 1.0 / jnp.maximum(deg_ref[...], 1.0)
    out = s_ref[...]
    wn = wn_ref[...]
    for p in range(4):
        out += jnp.dot(agg_ref[p] * inv, wn[64 * p:64 * (p + 1), :],
                       preferred_element_type=jnp.float32)
    out_ref[...] = jnp.maximum(out, 0.0)


@functools.lru_cache(maxsize=None)
def _tc_combine():
    return pl.pallas_call(
        _tc_combine_body,
        grid=(N_NODES // BM,),
        in_specs=[
            pl.BlockSpec((BM, D), lambda i: (i, 0)),
            pl.BlockSpec((4, BM, 64), lambda i: (0, i, 0)),
            pl.BlockSpec((BM, 1), lambda i: (i, 0)),
            pl.BlockSpec((D, D), lambda i: (0, 0)),
        ],
        out_specs=pl.BlockSpec((BM, D), lambda i: (i, 0)),
        out_shape=jax.ShapeDtypeStruct((N_NODES, D), jnp.float32),
    )


def kernel(features, edge_index, W_self1, W_neigh1, b1, W_self2, W_neigh2, b2):
    src = edge_index[0].astype(jnp.int32)
    dst = edge_index[1].astype(jnp.int32)
    srcidx = (src * 4).reshape(N_TILES, N_CHUNKS, CHUNK)
    dstidx = dst.reshape(N_TILES, N_CHUNKS, CHUNK)

    tc_self = _tc_self()
    tc_combine = _tc_combine()
    agg1, deg = _sc_aggregate_l1()(
        features.reshape(4 * N_NODES, 64), srcidx, dstidx)
    s1 = tc_self(features, W_self1, b1.reshape(1, D))
    deg2d = deg[:N_NODES].reshape(N_NODES, 1)
    h1 = tc_combine(s1, agg1, deg2d, W_neigh1)
    agg2 = _sc_aggregate_l2()(
        h1.reshape(4 * N_NODES, 64), srcidx, dstidx)
    s2 = tc_self(h1, W_self2, b2.reshape(1, D))
    h2 = tc_combine(s2, agg2, deg2d, W_neigh2)
    return h2
